# baseline (speedup 1.0000x reference)
import jax
import jax.numpy as jnp
from jax.experimental import pallas as pl
from jax.experimental.pallas import tpu as pltpu

D_MODEL = 1024
HIDDEN = 1024
NUM_EXPERTS = 64
TOKENS = 128


def _router_body(x_ref, wr_ref, br_ref, gates_ref, idx2_ref, act_ref, cnt_ref):
    xv = x_ref[...]
    logits = jax.lax.dot_general(wr_ref[...], xv, (((0,), (1,)), ((), ())),
                                 preferred_element_type=jnp.float32)
    logits = logits + br_ref[...]
    mx = jnp.max(logits, axis=0, keepdims=True)
    p = jnp.exp(logits - mx)
    probs = p / jnp.sum(p, axis=0, keepdims=True)
    iota_e = jax.lax.broadcasted_iota(jnp.int32, probs.shape, 0)
    m1 = jnp.max(probs, axis=0, keepdims=True)
    i1 = jnp.min(jnp.where(probs == m1, iota_e, NUM_EXPERTS),
                 axis=0, keepdims=True)
    sel1 = iota_e == i1
    p2 = jnp.where(sel1, -jnp.inf, probs)
    m2 = jnp.max(p2, axis=0, keepdims=True)
    i2 = jnp.min(jnp.where(p2 == m2, iota_e, NUM_EXPERTS),
                 axis=0, keepdims=True)
    sel2 = iota_e == i2
    den = m1 + m2
    gates = (jnp.where(sel1, m1, 0.0) + jnp.where(sel2, m2, 0.0)) / den
    gates_ref[...] = gates
    idx2_ref[...] = jnp.concatenate([i1, i2], axis=0)

    maskf = jnp.max(jnp.where(gates > 0.0, 1.0, 0.0), axis=1, keepdims=True)
    cnt = jnp.sum(maskf)
    tri = (jax.lax.broadcasted_iota(jnp.int32, (NUM_EXPERTS, NUM_EXPERTS), 0)
           >= jax.lax.broadcasted_iota(jnp.int32, (NUM_EXPERTS, NUM_EXPERTS), 1)
           ).astype(jnp.float32)
    pos = jnp.dot(tri, maskf, preferred_element_type=jnp.float32) - 1.0
    iota_s = jax.lax.broadcasted_iota(
        jnp.int32, (NUM_EXPERTS, NUM_EXPERTS), 1).astype(jnp.float32)
    onehot = jnp.where((iota_s == pos) & (maskf > 0.0), 1.0, 0.0)
    e_idsf = jax.lax.broadcasted_iota(
        jnp.int32, (NUM_EXPERTS, 1), 0).astype(jnp.float32)
    act = jax.lax.dot_general(onehot, e_idsf, (((0,), (0,)), ((), ())),
                              preferred_element_type=jnp.float32)
    last = jnp.max(act)
    slot_iota = jax.lax.broadcasted_iota(
        jnp.int32, (NUM_EXPERTS, 1), 0).astype(jnp.float32)
    act = jnp.where(slot_iota < cnt, act, last)
    act_ref[...] = act.astype(jnp.int32)
    cnt_ref[...] = jnp.full((1, 1), cnt, jnp.float32).astype(jnp.int32)


def _main_body(act_s, cnt_s, x_ref, ws1_ref, ws2_ref, scp_ref,
               w1_ref, w2_ref, cp_ref, g_ref, out_ref):
    i = pl.program_id(0)
    xv = x_ref[...]

    @pl.when(i == 0)
    def _init_and_shared():
        sh = jax.nn.silu(jnp.dot(xv, ws1_ref[...],
                                 preferred_element_type=jnp.float32))
        sh = sh * jnp.dot(xv, ws2_ref[...], preferred_element_type=jnp.float32)
        out_ref[...] = jnp.dot(sh, scp_ref[...],
                               preferred_element_type=jnp.float32)

    @pl.when(i < cnt_s[0])
    def _expert():
        g = g_ref[0]
        h = jax.nn.silu(jnp.dot(xv, w1_ref[0],
                                preferred_element_type=jnp.float32))
        h = h * jnp.dot(xv, w2_ref[0], preferred_element_type=jnp.float32)
        out_ref[...] += jnp.dot(h * g, cp_ref[0],
                                preferred_element_type=jnp.float32)


@jax.jit
def kernel(x, Ws1, Ws2, Scp, W1, W2, Cp, Wr, br):
    br2 = br.reshape(NUM_EXPERTS, 1)
    gates, idx2, act, cnt = pl.pallas_call(
        _router_body,
        grid=(1,),
        in_specs=[
            pl.BlockSpec((TOKENS, D_MODEL), lambda i: (0, 0)),
            pl.BlockSpec((D_MODEL, NUM_EXPERTS), lambda i: (0, 0)),
            pl.BlockSpec((NUM_EXPERTS, 1), lambda i: (0, 0)),
        ],
        out_specs=[
            pl.BlockSpec((NUM_EXPERTS, TOKENS), lambda i: (0, 0)),
            pl.BlockSpec((2, TOKENS), lambda i: (0, 0)),
            pl.BlockSpec((NUM_EXPERTS, 1), lambda i: (0, 0)),
            pl.BlockSpec((1, 1), lambda i: (0, 0)),
        ],
        out_shape=[
            jax.ShapeDtypeStruct((NUM_EXPERTS, TOKENS), jnp.float32),
            jax.ShapeDtypeStruct((2, TOKENS), jnp.int32),
            jax.ShapeDtypeStruct((NUM_EXPERTS, 1), jnp.int32),
            jax.ShapeDtypeStruct((1, 1), jnp.int32),
        ],
    )(x, Wr, br2)

    gates3 = gates.reshape(NUM_EXPERTS, TOKENS, 1)
    active = act.reshape(NUM_EXPERTS)
    count = cnt.reshape(1)

    grid_spec = pltpu.PrefetchScalarGridSpec(
        num_scalar_prefetch=2,
        grid=(NUM_EXPERTS,),
        in_specs=[
            pl.BlockSpec((TOKENS, D_MODEL), lambda i, a, c: (0, 0)),
            pl.BlockSpec((D_MODEL, HIDDEN), lambda i, a, c: (0, 0)),
            pl.BlockSpec((D_MODEL, HIDDEN), lambda i, a, c: (0, 0)),
            pl.BlockSpec((HIDDEN, D_MODEL), lambda i, a, c: (0, 0)),
            pl.BlockSpec((1, D_MODEL, HIDDEN), lambda i, a, c: (a[i], 0, 0)),
            pl.BlockSpec((1, D_MODEL, HIDDEN), lambda i, a, c: (a[i], 0, 0)),
            pl.BlockSpec((1, HIDDEN, D_MODEL), lambda i, a, c: (a[i], 0, 0)),
            pl.BlockSpec((1, TOKENS, 1), lambda i, a, c: (a[i], 0, 0)),
        ],
        out_specs=pl.BlockSpec((TOKENS, D_MODEL), lambda i, a, c: (0, 0)),
    )
    out = pl.pallas_call(
        _main_body,
        grid_spec=grid_spec,
        out_shape=jax.ShapeDtypeStruct((TOKENS, D_MODEL), jnp.float32),
        compiler_params=pltpu.CompilerParams(
            dimension_semantics=("arbitrary",),
            vmem_limit_bytes=100 * 1024 * 1024,
        ),
    )(active, count, x, Ws1, Ws2, Scp, W1, W2, Cp, gates3)
    return out


# speedup vs baseline: 1.0487x; 1.0487x over previous
import jax
import jax.numpy as jnp
from jax.experimental import pallas as pl
from jax.experimental.pallas import tpu as pltpu

D_MODEL = 1024
HIDDEN = 1024
NUM_EXPERTS = 64
TOKENS = 128


def _moe_body(x_ref, ws1_ref, ws2_ref, scp_ref, w1_ref, w2_ref, cp_ref,
              wr_ref, br_ref, out_ref,
              m1_ref, m2_ref, i1_ref, i2_ref, den_ref):
    e = pl.program_id(0)
    xv = x_ref[...]

    @pl.when(e == 0)
    def _router_init_and_shared():
        logits = jnp.dot(xv, wr_ref[...], preferred_element_type=jnp.float32)
        logits = logits + br_ref[...]
        probs = jax.nn.softmax(logits, axis=-1)
        iota = jax.lax.broadcasted_iota(jnp.int32, probs.shape, 1)
        m1 = jnp.max(probs, axis=-1, keepdims=True)
        i1 = jnp.min(jnp.where(probs == m1, iota, NUM_EXPERTS),
                     axis=-1, keepdims=True)
        p2 = jnp.where(iota == i1, -jnp.inf, probs)
        m2 = jnp.max(p2, axis=-1, keepdims=True)
        i2 = jnp.min(jnp.where(p2 == m2, iota, NUM_EXPERTS),
                     axis=-1, keepdims=True)
        m1_ref[...] = m1
        m2_ref[...] = m2
        i1_ref[...] = i1
        i2_ref[...] = i2
        den_ref[...] = m1 + m2
        sh = jax.nn.silu(jnp.dot(xv, ws1_ref[...],
                                 preferred_element_type=jnp.float32))
        sh = sh * jnp.dot(xv, ws2_ref[...], preferred_element_type=jnp.float32)
        out_ref[...] = jnp.dot(sh, scp_ref[...],
                               preferred_element_type=jnp.float32)

    g = (jnp.where(i1_ref[...] == e, m1_ref[...], 0.0)
         + jnp.where(i2_ref[...] == e, m2_ref[...], 0.0)) / den_ref[...]

    h = jax.nn.silu(jnp.dot(xv, w1_ref[0], preferred_element_type=jnp.float32))
    h = h * jnp.dot(xv, w2_ref[0], preferred_element_type=jnp.float32)
    out_ref[...] += jnp.dot(h * g, cp_ref[0], preferred_element_type=jnp.float32)


@jax.jit
def kernel(x, Ws1, Ws2, Scp, W1, W2, Cp, Wr, br):
    br2 = br.reshape(1, NUM_EXPERTS)
    out = pl.pallas_call(
        _moe_body,
        grid=(NUM_EXPERTS,),
        in_specs=[
            pl.BlockSpec((TOKENS, D_MODEL), lambda e: (0, 0)),
            pl.BlockSpec((D_MODEL, HIDDEN), lambda e: (0, 0)),
            pl.BlockSpec((D_MODEL, HIDDEN), lambda e: (0, 0)),
            pl.BlockSpec((HIDDEN, D_MODEL), lambda e: (0, 0)),
            pl.BlockSpec((1, D_MODEL, HIDDEN), lambda e: (e, 0, 0)),
            pl.BlockSpec((1, D_MODEL, HIDDEN), lambda e: (e, 0, 0)),
            pl.BlockSpec((1, HIDDEN, D_MODEL), lambda e: (e, 0, 0)),
            pl.BlockSpec((D_MODEL, NUM_EXPERTS), lambda e: (0, 0)),
            pl.BlockSpec((1, NUM_EXPERTS), lambda e: (0, 0)),
        ],
        out_specs=pl.BlockSpec((TOKENS, D_MODEL), lambda e: (0, 0)),
        out_shape=jax.ShapeDtypeStruct((TOKENS, D_MODEL), jnp.float32),
        scratch_shapes=[
            pltpu.VMEM((TOKENS, 1), jnp.float32),
            pltpu.VMEM((TOKENS, 1), jnp.float32),
            pltpu.VMEM((TOKENS, 1), jnp.int32),
            pltpu.VMEM((TOKENS, 1), jnp.int32),
            pltpu.VMEM((TOKENS, 1), jnp.float32),
        ],
        compiler_params=pltpu.CompilerParams(
            dimension_semantics=("arbitrary",),
            vmem_limit_bytes=100 * 1024 * 1024,
        ),
    )(x, Ws1, Ws2, Scp, W1, W2, Cp, Wr, br2)
    return out


# speedup vs baseline: 1.0575x; 1.0084x over previous
import jax
import jax.numpy as jnp
from jax.experimental import pallas as pl
from jax.experimental.pallas import tpu as pltpu

D_MODEL = 1024
HIDDEN = 1024
NUM_EXPERTS = 64
TOKENS = 128
SH_STEPS = 8
SH_CHUNK = HIDDEN // SH_STEPS


def _moe_body(x_ref, ws1_ref, ws2_ref, scp_ref, w1_ref, w2_ref, cp_ref,
              wr_ref, br_ref, out_ref,
              m1_ref, m2_ref, i1_ref, i2_ref, den_ref):
    e = pl.program_id(0)
    xv = x_ref[...]

    @pl.when(e == 0)
    def _router_init_and_shared():
        logits = jnp.dot(xv, wr_ref[...], preferred_element_type=jnp.float32)
        logits = logits + br_ref[...]
        probs = jax.nn.softmax(logits, axis=-1)
        iota = jax.lax.broadcasted_iota(jnp.int32, probs.shape, 1)
        m1 = jnp.max(probs, axis=-1, keepdims=True)
        i1 = jnp.min(jnp.where(probs == m1, iota, NUM_EXPERTS),
                     axis=-1, keepdims=True)
        p2 = jnp.where(iota == i1, -jnp.inf, probs)
        m2 = jnp.max(p2, axis=-1, keepdims=True)
        i2 = jnp.min(jnp.where(p2 == m2, iota, NUM_EXPERTS),
                     axis=-1, keepdims=True)
        m1_ref[...] = m1
        m2_ref[...] = m2
        i1_ref[...] = i1
        i2_ref[...] = i2
        den_ref[...] = m1 + m2
        out_ref[...] = jnp.zeros_like(out_ref)

    @pl.when(e < SH_STEPS)
    def _shared_chunk():
        sh = jax.nn.silu(jnp.dot(xv, ws1_ref[...],
                                 preferred_element_type=jnp.float32))
        sh = sh * jnp.dot(xv, ws2_ref[...], preferred_element_type=jnp.float32)
        out_ref[...] += jnp.dot(sh, scp_ref[...],
                                preferred_element_type=jnp.float32)

    g = (jnp.where(i1_ref[...] == e, m1_ref[...], 0.0)
         + jnp.where(i2_ref[...] == e, m2_ref[...], 0.0)) / den_ref[...]

    h = jax.nn.silu(jnp.dot(xv, w1_ref[0], preferred_element_type=jnp.float32))
    h = h * jnp.dot(xv, w2_ref[0], preferred_element_type=jnp.float32)
    out_ref[...] += jnp.dot(h * g, cp_ref[0], preferred_element_type=jnp.float32)


@jax.jit
def kernel(x, Ws1, Ws2, Scp, W1, W2, Cp, Wr, br):
    br2 = br.reshape(1, NUM_EXPERTS)
    out = pl.pallas_call(
        _moe_body,
        grid=(NUM_EXPERTS,),
        in_specs=[
            pl.BlockSpec((TOKENS, D_MODEL), lambda e: (0, 0)),
            pl.BlockSpec((D_MODEL, SH_CHUNK),
                         lambda e: (0, jnp.minimum(e, SH_STEPS - 1))),
            pl.BlockSpec((D_MODEL, SH_CHUNK),
                         lambda e: (0, jnp.minimum(e, SH_STEPS - 1))),
            pl.BlockSpec((SH_CHUNK, D_MODEL),
                         lambda e: (jnp.minimum(e, SH_STEPS - 1), 0)),
            pl.BlockSpec((1, D_MODEL, HIDDEN), lambda e: (e, 0, 0)),
            pl.BlockSpec((1, D_MODEL, HIDDEN), lambda e: (e, 0, 0)),
            pl.BlockSpec((1, HIDDEN, D_MODEL), lambda e: (e, 0, 0)),
            pl.BlockSpec((D_MODEL, NUM_EXPERTS), lambda e: (0, 0)),
            pl.BlockSpec((1, NUM_EXPERTS), lambda e: (0, 0)),
        ],
        out_specs=pl.BlockSpec((TOKENS, D_MODEL), lambda e: (0, 0)),
        out_shape=jax.ShapeDtypeStruct((TOKENS, D_MODEL), jnp.float32),
        scratch_shapes=[
            pltpu.VMEM((TOKENS, 1), jnp.float32),
            pltpu.VMEM((TOKENS, 1), jnp.float32),
            pltpu.VMEM((TOKENS, 1), jnp.int32),
            pltpu.VMEM((TOKENS, 1), jnp.int32),
            pltpu.VMEM((TOKENS, 1), jnp.float32),
        ],
        compiler_params=pltpu.CompilerParams(
            dimension_semantics=("arbitrary",),
            vmem_limit_bytes=100 * 1024 * 1024,
        ),
    )(x, Ws1, Ws2, Scp, W1, W2, Cp, Wr, br2)
    return out
